# Initial kernel scaffold; baseline (speedup 1.0000x reference)
import jax
import jax.numpy as jnp
from jax.experimental import pallas as pl


def kernel(x, classifier_weight, dynamic_threshs):
    raise NotImplementedError("write your pallas kernel here")


# speedup vs baseline: 84.5764x; 84.5764x over previous
import functools

import jax
import jax.numpy as jnp
from jax import lax
from jax.experimental import pallas as pl
from jax.experimental.pallas import tpu as pltpu
from jax.experimental.pallas import tpu_sc as plsc

_B = 1024
_F = 64
_N = 100000
_CBLK = 2048
_NBLK = (_N + _CBLK - 1) // _CBLK
_NEG = -3.0e38
_BIGI = 2**31 - 1


def _main_body(x_ref, w_ref, th_ref, stats_ref, idx_ref,
               m_s, z_s, s1_s, t_s, pred_s, v0_s, v1_s, v2_s,
               i0_s, i1_s, i2_s):
    blk = pl.program_id(0)

    @pl.when(blk == 0)
    def _():
        m_s[...] = jnp.full_like(m_s, _NEG)
        z_s[...] = jnp.zeros_like(z_s)
        s1_s[...] = jnp.zeros_like(s1_s)
        t_s[...] = jnp.zeros_like(t_s)
        pred_s[...] = jnp.zeros_like(pred_s)
        v0_s[...] = jnp.full_like(v0_s, _NEG)
        v1_s[...] = jnp.full_like(v1_s, _NEG)
        v2_s[...] = jnp.full_like(v2_s, _NEG)
        i0_s[...] = jnp.zeros_like(i0_s)
        i1_s[...] = jnp.zeros_like(i1_s)
        i2_s[...] = jnp.zeros_like(i2_s)

    x = x_ref[...]
    w = w_ref[...]
    th = th_ref[0]

    L = lax.dot_general(x, w, (((1,), (1,)), ((), ())),
                        preferred_element_type=jnp.float32)
    col = lax.broadcasted_iota(jnp.int32, (_B, _CBLK), 1)
    valid = (blk * _CBLK + col) < _N
    L = jnp.where(valid, L, _NEG)

    bm = jnp.max(L, axis=1, keepdims=True)
    m_old = m_s[...]
    m_new = jnp.maximum(m_old, bm)
    corr = jnp.exp(m_old - m_new)
    e = jnp.exp(L - m_new)
    z_s[...] = z_s[...] * corr + jnp.sum(e, axis=1, keepdims=True)
    s1_s[...] = s1_s[...] * corr + jnp.sum(e * L, axis=1, keepdims=True)
    m_s[...] = m_new

    is_bm = (L == bm) & valid
    bai = jnp.min(jnp.where(is_bm, col, _BIGI), axis=1, keepdims=True)
    onehot = col == bai
    t_bm = jnp.sum(jnp.where(onehot, jnp.broadcast_to(th, (_B, _CBLK)), 0.0),
                   axis=1, keepdims=True)
    upd = bm > m_old
    pred_s[...] = jnp.where(upd, blk * _CBLK + bai, pred_s[...])
    t_s[...] = jnp.where(upd, t_bm, t_s[...])

    p2 = jnp.sum(w * w, axis=1)[None, :]
    s = jnp.where(valid, L - 0.5 * p2, _NEG)
    v0, v1, v2 = v0_s[...], v1_s[...], v2_s[...]
    i0, i1, i2 = i0_s[...], i1_s[...], i2_s[...]
    for _ in range(3):
        bv = jnp.max(s, axis=1, keepdims=True)
        bi = jnp.min(jnp.where(s == bv, col, _BIGI), axis=1, keepdims=True)
        bgi = blk * _CBLK + bi
        s = jnp.where(col == bi, _NEG, s)
        gt0 = bv > v0
        gt1 = bv > v1
        gt2 = bv > v2
        v0, i0, v1, i1, v2, i2 = (
            jnp.where(gt0, bv, v0),
            jnp.where(gt0, bgi, i0),
            jnp.where(gt0, v0, jnp.where(gt1, bv, v1)),
            jnp.where(gt0, i0, jnp.where(gt1, bgi, i1)),
            jnp.where(gt0 | gt1, v1, jnp.where(gt2, bv, v2)),
            jnp.where(gt0 | gt1, i1, jnp.where(gt2, bgi, i2)),
        )
    v0_s[...], v1_s[...], v2_s[...] = v0, v1, v2
    i0_s[...], i1_s[...], i2_s[...] = i0, i1, i2

    @pl.when(blk == _NBLK - 1)
    def _():
        stats_ref[...] = jnp.concatenate(
            [m_s[...], z_s[...], s1_s[...], t_s[...]], axis=1)
        idx_ref[...] = jnp.concatenate(
            [pred_s[...], i0_s[...], i1_s[...], i2_s[...]], axis=1)


_main_call = pl.pallas_call(
    _main_body,
    grid=(_NBLK,),
    in_specs=[
        pl.BlockSpec((_B, _F), lambda i: (0, 0)),
        pl.BlockSpec((_CBLK, _F), lambda i: (i, 0)),
        pl.BlockSpec((1, 1, _CBLK), lambda i: (i, 0, 0)),
    ],
    out_specs=[
        pl.BlockSpec((_B, 4), lambda i: (0, 0)),
        pl.BlockSpec((_B, 4), lambda i: (0, 0)),
    ],
    out_shape=[
        jax.ShapeDtypeStruct((_B, 4), jnp.float32),
        jax.ShapeDtypeStruct((_B, 4), jnp.int32),
    ],
    scratch_shapes=(
        [pltpu.VMEM((_B, 1), jnp.float32) for _ in range(4)]
        + [pltpu.VMEM((_B, 1), jnp.int32)]
        + [pltpu.VMEM((_B, 1), jnp.float32) for _ in range(3)]
        + [pltpu.VMEM((_B, 1), jnp.int32) for _ in range(3)]
    ),
    compiler_params=pltpu.CompilerParams(
        dimension_semantics=("arbitrary",)),
)


@functools.lru_cache(maxsize=1)
def _make_sc_gather():
    info = plsc.get_sparse_core_info()
    nw = info.num_cores * info.num_subcores
    rpw = _B // nw

    mesh = plsc.VectorSubcoreMesh(core_axis_name="c", subcore_axis_name="s")

    @functools.partial(
        pl.kernel, mesh=mesh,
        out_type=[jax.ShapeDtypeStruct((_B, _F), jnp.float32)
                  for _ in range(3)],
        scratch_types=[
            pltpu.VMEM((rpw,), jnp.int32),
            pltpu.VMEM((rpw, _F), jnp.float32),
            pltpu.SemaphoreType.DMA,
        ],
        compiler_params=pltpu.CompilerParams(use_tc_tiling_on_sc=False),
    )
    def sc_gather(w_hbm, c0_hbm, c1_hbm, c2_hbm, o0, o1, o2,
                  idx_v, rows_v, sem):
        wid = lax.axis_index("s") * info.num_cores + lax.axis_index("c")
        base = wid * rpw
        for c_hbm, o_hbm in ((c0_hbm, o0), (c1_hbm, o1), (c2_hbm, o2)):
            pltpu.sync_copy(c_hbm.at[pl.ds(base, rpw)], idx_v)
            pltpu.async_copy(w_hbm.at[idx_v], rows_v, sem).wait()
            pltpu.sync_copy(rows_v, o_hbm.at[pl.ds(base, rpw)])

    return sc_gather


def _epi_body(stats_ref, x_ref, m0_ref, m1_ref, m2_ref, out_ref):
    st = stats_ref[...]
    m, z, s1, t = st[:, 0:1], st[:, 1:2], st[:, 2:3], st[:, 3:4]
    x = x_ref[...]

    logz = m + jnp.log(z)
    ent_full = logz - s1 / z
    max_val = jnp.exp(m - logz)
    reliable = (max_val >= t).astype(jnp.float32)
    ent = reliable * ent_full + (1.0 - reliable) * jnp.log(jnp.float32(_N))

    def mahal(mu):
        diff = (x - mu) * (1.0 / 0.001)
        n = jnp.sqrt(jnp.sum(diff * diff, axis=1, keepdims=True))
        dn = diff / jnp.maximum(n, 1e-12)
        return jnp.sqrt(jnp.sum(dn * dn, axis=1, keepdims=True))

    d0 = mahal(m0_ref[...])
    d1 = mahal(m1_ref[...])
    d2 = mahal(m2_ref[...])
    min_d = jnp.minimum(jnp.minimum(d0, d1), d2)
    sims_min = jnp.exp(-min_d)
    sims_sum = jnp.exp(-d0) + jnp.exp(-d1) + jnp.exp(-d2)
    pcl = -jnp.log(sims_min / sims_sum)
    out_ref[...] = ent + pcl


_epi_call = pl.pallas_call(
    _epi_body,
    out_shape=jax.ShapeDtypeStruct((_B, 1), jnp.float32),
)


def kernel(x, classifier_weight, dynamic_threshs):
    w = classifier_weight
    th3 = jnp.pad(dynamic_threshs,
                  (0, _NBLK * _CBLK - _N)).reshape(_NBLK, 1, _CBLK)
    stats, idx4 = _main_call(x, w, th3)
    c0, c1, c2 = idx4[:, 1], idx4[:, 2], idx4[:, 3]
    mu0, mu1, mu2 = _make_sc_gather()(w, c0, c1, c2)
    out = _epi_call(stats, x, mu0, mu1, mu2)
    return out.reshape(_B)
